# baseline (speedup 1.0000x reference)
import functools

import jax
import jax.numpy as jnp
from jax import lax
from jax.experimental import pallas as pl
from jax.experimental.pallas import tpu as pltpu
from jax.experimental.pallas import tpu_sc as plsc

N = 10000
E = 320000
D = 128
DE = 16
EMB = 9
DIM_OUT = 10

K = 128
NCHUNK = E // K
NW = 32
ROWS_MAIN = 632
ROWS_TAIL = N - 15 * ROWS_MAIN


_sc_mesh = plsc.VectorSubcoreMesh(core_axis_name="c", subcore_axis_name="s")


NCH_MIN = NCHUNK // NW
NCH_REM = NCHUNK - NCH_MIN * NW
NGROUPS = (NCH_MIN + 1 + 2) // 3 + 1


@functools.partial(
    pl.kernel,
    out_type=jax.ShapeDtypeStruct((2 * N, D), jnp.float32),
    mesh=_sc_mesh,
    scratch_types=[
        [pltpu.VMEM((K, D), jnp.float32) for _ in range(3)],
        [pltpu.VMEM((K,), jnp.int32) for _ in range(3)],
        [pltpu.VMEM((K,), jnp.int32) for _ in range(3)],
        pltpu.VMEM_SHARED((N, D), jnp.float32),
        [pltpu.SemaphoreType.DMA for _ in range(3)],
        [pltpu.SemaphoreType.DMA for _ in range(3)],
    ],
)
def _edge_pass(h_hbm, econ_hbm, src_hbm, dst_hbm, out_hbm,
               buf, srci_v, dsti_v, agg_sp, sem, scsem):
    cid = lax.axis_index("c")
    sid = lax.axis_index("s")
    wid = sid * 2 + cid
    cb = wid * NCH_MIN + jnp.minimum(wid, NCH_REM)
    nch = NCH_MIN + jnp.where(wid < NCH_REM, 1, 0)

    zero = jnp.zeros((16,), jnp.float32)

    def zero_row(i, _):
        for k in range(8):
            buf[0][i, pl.ds(k * 16, 16)] = zero
        return 0

    lax.fori_loop(0, K, zero_row, 0)

    def _zero_slice(base, sizes):
        off = 0
        for sz in sizes:
            pltpu.sync_copy(buf[0].at[pl.ds(0, sz)],
                            agg_sp.at[pl.ds(base + off, sz)])
            off += sz

    @pl.when(sid < 15)
    def _():
        _zero_slice(sid * ROWS_MAIN, [128, 128, 128, 128, 120])

    @pl.when(sid == 15)
    def _():
        _zero_slice(15 * ROWS_MAIN, [128, 128, 128, 128, 8])

    plsc.subcore_barrier()

    def issue_loads(jj, s):
        pltpu.async_copy(econ_hbm.at[pl.ds((cb + jj) * K, K)], buf[s], sem[s])
        pltpu.async_copy(src_hbm.at[pl.ds((cb + jj) * K, K)], srci_v[s], sem[s])
        pltpu.async_copy(dst_hbm.at[pl.ds((cb + jj) * K, K)], dsti_v[s], sem[s])

    def wait_loads(s):
        pltpu.make_async_copy(econ_hbm.at[pl.ds(0, K)], buf[s], sem[s]).wait()
        pltpu.make_async_copy(src_hbm.at[pl.ds(0, K)], srci_v[s], sem[s]).wait()
        pltpu.make_async_copy(dst_hbm.at[pl.ds(0, K)], dsti_v[s], sem[s]).wait()

    def wait_gather(s):
        pltpu.make_async_copy(econ_hbm.at[pl.ds(0, K)], buf[s], sem[s]).wait()

    def wait_scatter(s):
        pltpu.make_async_copy(buf[s], agg_sp.at[pl.ds(0, K)], scsem[s]).wait()

    pltpu.sync_copy(econ_hbm.at[pl.ds(cb * K, K)], buf[0])
    pltpu.sync_copy(src_hbm.at[pl.ds(cb * K, K)], srci_v[0])
    pltpu.sync_copy(dst_hbm.at[pl.ds(cb * K, K)], dsti_v[0])
    pltpu.async_copy(h_hbm.at[srci_v[0]], buf[0], sem[0], add=True)
    issue_loads(1, 1)

    def group(g, _):
        for u in range(3):
            j = g * 3 + u
            s1 = (u + 1) % 3
            s2 = (u + 2) % 3

            @pl.when(j + 1 < nch)
            def _():
                wait_loads(s1)
                pltpu.async_copy(h_hbm.at[srci_v[s1]], buf[s1], sem[s1],
                                 add=True)

            @pl.when(j < nch)
            def _():
                wait_gather(u)

                def msg_row(i, _):
                    for r in range(4):
                        for k in range(8):
                            sl = pl.ds(k * 16, 16)
                            buf[u][i * 4 + r, sl] = jnp.maximum(
                                buf[u][i * 4 + r, sl], 0.0)
                    return 0

                lax.fori_loop(0, K // 4, msg_row, 0)
                pltpu.async_copy(buf[u], agg_sp.at[dsti_v[u]], scsem[u],
                                 add=True)

            @pl.when(jnp.logical_and(j >= 1, j + 2 < nch))
            def _():
                wait_scatter(s2)

            @pl.when(j + 2 < nch)
            def _():
                issue_loads(j + 2, s2)
        return 0

    lax.fori_loop(0, NGROUPS, group, 0)

    @pl.when(wid < NCH_REM)
    def _():
        for c in (NCH_MIN - 2, NCH_MIN - 1, NCH_MIN):
            wait_scatter(c % 3)

    @pl.when(wid >= NCH_REM)
    def _():
        for c in (NCH_MIN - 3, NCH_MIN - 2, NCH_MIN - 1):
            wait_scatter(c % 3)

    plsc.subcore_barrier()

    @pl.when(sid < 15)
    def _():
        row0 = sid * ROWS_MAIN
        pltpu.sync_copy(agg_sp.at[pl.ds(row0, ROWS_MAIN)],
                        out_hbm.at[pl.ds(cid * N + row0, ROWS_MAIN)])

    @pl.when(sid == 15)
    def _():
        row0 = 15 * ROWS_MAIN
        pltpu.sync_copy(agg_sp.at[pl.ds(row0, ROWS_TAIL)],
                        out_hbm.at[pl.ds(cid * N + row0, ROWS_TAIL)])


_RT_N = 1000
_GRID_N = N // _RT_N
_RT_E = 2000
_GRID_E = E // _RT_E


def _node_init_body(x_ref, ploop_ref, wn_ref, bn_ref, wl0_ref, h_ref):
    h = jnp.dot(x_ref[...], wn_ref[...], preferred_element_type=jnp.float32)
    h += jnp.dot(ploop_ref[...], wl0_ref[...], preferred_element_type=jnp.float32)
    h_ref[...] = h + bn_ref[...]


def _node_init(x, poly_loop, W_node, b_node, W_loop0):
    return pl.pallas_call(
        _node_init_body,
        grid=(_GRID_N,),
        in_specs=[
            pl.BlockSpec((_RT_N, D), lambda i: (i, 0)),
            pl.BlockSpec((_RT_N, EMB), lambda i: (i, 0)),
            pl.BlockSpec((D, D), lambda i: (0, 0)),
            pl.BlockSpec((1, D), lambda i: (0, 0)),
            pl.BlockSpec((EMB, D), lambda i: (0, 0)),
        ],
        out_specs=pl.BlockSpec((_RT_N, D), lambda i: (i, 0)),
        out_shape=jax.ShapeDtypeStruct((N, D), jnp.float32),
    )(x, poly_loop, W_node, b_node, W_loop0)


def _econ0_body(ea_ref, pc_ref, we_ref, be_ref, wc_ref, e0_ref):
    pc = pc_ref[...]
    base = jnp.dot(ea_ref[...], we_ref[...], preferred_element_type=jnp.float32)
    base += be_ref[...]
    p0 = jnp.dot(pc, wc_ref[0], preferred_element_type=jnp.float32)
    m0 = (pc[:, 1] != 0).astype(jnp.float32)[:, None]
    e0_ref[...] = base + m0 * p0


def _econ12_body(ea_ref, pc_ref, we_ref, be_ref, wc_ref, e1_ref, e2_ref):
    pc = pc_ref[...]
    base = jnp.dot(ea_ref[...], we_ref[...], preferred_element_type=jnp.float32)
    base += be_ref[...]
    p0 = jnp.dot(pc, wc_ref[0], preferred_element_type=jnp.float32)
    p1 = jnp.dot(pc, wc_ref[1], preferred_element_type=jnp.float32)
    p2 = jnp.dot(pc, wc_ref[2], preferred_element_type=jnp.float32)
    m0 = (pc[:, 1] != 0).astype(jnp.float32)[:, None]
    m1 = (pc[:, 2] != 0).astype(jnp.float32)[:, None]
    e1 = base + m0 * p0 + m1 * p1
    e1_ref[...] = e1
    e2_ref[...] = e1 + p2


_ECON_SPECS = [
    pl.BlockSpec((_RT_E, DE), lambda i: (i, 0)),
    pl.BlockSpec((_RT_E, EMB), lambda i: (i, 0)),
    pl.BlockSpec((DE, D), lambda i: (0, 0)),
    pl.BlockSpec((1, D), lambda i: (0, 0)),
    pl.BlockSpec((3, EMB, D), lambda i: (0, 0, 0)),
]


def _econ0(edge_attr, poly_conn, W_edge, b_edge, W_conn):
    out = jax.ShapeDtypeStruct((E, D), jnp.float32)
    return pl.pallas_call(
        _econ0_body,
        grid=(_GRID_E,),
        in_specs=_ECON_SPECS,
        out_specs=pl.BlockSpec((_RT_E, D), lambda i: (i, 0)),
        out_shape=out,
    )(edge_attr, poly_conn, W_edge, b_edge, W_conn)


def _econ12(edge_attr, poly_conn, W_edge, b_edge, W_conn):
    out = jax.ShapeDtypeStruct((E, D), jnp.float32)
    return pl.pallas_call(
        _econ12_body,
        grid=(_GRID_E,),
        in_specs=_ECON_SPECS,
        out_specs=[pl.BlockSpec((_RT_E, D), lambda i: (i, 0))] * 2,
        out_shape=[out, out],
    )(edge_attr, poly_conn, W_edge, b_edge, W_conn)


def _update_body(h_ref, agg_ref, w_ref, b_ref, ploop_ref, wl_ref, o_ref):
    agg = agg_ref[0] + agg_ref[1]
    u = jnp.dot(agg, w_ref[...], preferred_element_type=jnp.float32) + b_ref[...]
    h = h_ref[...] + jnp.maximum(u, 0.0)
    h += jnp.dot(ploop_ref[...], wl_ref[...], preferred_element_type=jnp.float32)
    o_ref[...] = h


def _update(h, agg2, W, b, poly_loop, wl):
    return pl.pallas_call(
        _update_body,
        grid=(_GRID_N,),
        in_specs=[
            pl.BlockSpec((_RT_N, D), lambda i: (i, 0)),
            pl.BlockSpec((2, _RT_N, D), lambda i: (0, i, 0)),
            pl.BlockSpec((D, D), lambda i: (0, 0)),
            pl.BlockSpec((1, D), lambda i: (0, 0)),
            pl.BlockSpec((_RT_N, EMB), lambda i: (i, 0)),
            pl.BlockSpec((EMB, D), lambda i: (0, 0)),
        ],
        out_specs=pl.BlockSpec((_RT_N, D), lambda i: (i, 0)),
        out_shape=jax.ShapeDtypeStruct((N, D), jnp.float32),
    )(h, agg2, W, b, poly_loop, wl)


def _final_body(h_ref, agg_ref, w_ref, b_ref, wo_ref, bo_ref, o_ref):
    agg = agg_ref[0] + agg_ref[1]
    u = jnp.dot(agg, w_ref[...], preferred_element_type=jnp.float32) + b_ref[...]
    h = h_ref[...] + jnp.maximum(u, 0.0)
    o_ref[...] = jnp.dot(h, wo_ref[...], preferred_element_type=jnp.float32) + bo_ref[...]


def _final(h, agg2, W, b, W_out, b_out):
    return pl.pallas_call(
        _final_body,
        grid=(_GRID_N,),
        in_specs=[
            pl.BlockSpec((_RT_N, D), lambda i: (i, 0)),
            pl.BlockSpec((2, _RT_N, D), lambda i: (0, i, 0)),
            pl.BlockSpec((D, D), lambda i: (0, 0)),
            pl.BlockSpec((1, D), lambda i: (0, 0)),
            pl.BlockSpec((D, DIM_OUT), lambda i: (0, 0)),
            pl.BlockSpec((1, DIM_OUT), lambda i: (0, 0)),
        ],
        out_specs=pl.BlockSpec((_RT_N, DIM_OUT), lambda i: (i, 0)),
        out_shape=jax.ShapeDtypeStruct((N, DIM_OUT), jnp.float32),
    )(h, agg2, W, b, W_out, b_out)


def kernel(x, edge_attr, edge_index, poly_loop, poly_index, poly_conn,
           W_node, b_node, W_edge, b_edge, W_loop, W_conn,
           W_msg, b_msg, W_out, b_out):
    del poly_index
    src = edge_index[0]
    dst = edge_index[1]

    h = _node_init(x, poly_loop, W_node, b_node.reshape(1, D), W_loop[0])
    e0 = _econ0(edge_attr, poly_conn, W_edge, b_edge.reshape(1, D), W_conn)

    agg = _edge_pass(h, e0, src, dst).reshape(2, N, D)
    e1, e2 = _econ12(edge_attr, poly_conn, W_edge, b_edge.reshape(1, D), W_conn)

    wl_zero = jnp.zeros((EMB, D), jnp.float32)
    wl_next = (None, W_loop[1], W_loop[2], None)
    econs = (None, e0, e1, e2)
    for r in range(4):
        if r > 0:
            agg = _edge_pass(h, econs[r], src, dst).reshape(2, N, D)
        if r < 3:
            wl = wl_next[r] if wl_next[r] is not None else wl_zero
            h = _update(h, agg, W_msg[r], b_msg[r].reshape(1, D), poly_loop, wl)
        else:
            out = _final(h, agg, W_msg[r], b_msg[r].reshape(1, D),
                         W_out, b_out.reshape(1, DIM_OUT))
    return out


# speedup vs baseline: 1.0373x; 1.0373x over previous
import functools

import jax
import jax.numpy as jnp
from jax import lax
from jax.experimental import pallas as pl
from jax.experimental.pallas import tpu as pltpu
from jax.experimental.pallas import tpu_sc as plsc

N = 10000
E = 320000
D = 128
DE = 16
EMB = 9
DIM_OUT = 10

K = 128
NCHUNK = E // K
NW = 32
ROWS_MAIN = 632
ROWS_TAIL = N - 15 * ROWS_MAIN


_sc_mesh = plsc.VectorSubcoreMesh(core_axis_name="c", subcore_axis_name="s")


NCH_MIN = NCHUNK // NW
NCH_REM = NCHUNK - NCH_MIN * NW
NGROUPS = (NCH_MIN + 1 + 2) // 3 + 1


@functools.partial(
    pl.kernel,
    out_type=jax.ShapeDtypeStruct((2 * N, D), jnp.float32),
    mesh=_sc_mesh,
    scratch_types=[
        [pltpu.VMEM((K, D), jnp.float32) for _ in range(3)],
        [pltpu.VMEM((K,), jnp.int32) for _ in range(3)],
        [pltpu.VMEM((K,), jnp.int32) for _ in range(3)],
        pltpu.VMEM_SHARED((N, D), jnp.float32),
        [pltpu.SemaphoreType.DMA for _ in range(3)],
        [pltpu.SemaphoreType.DMA for _ in range(3)],
    ],
)
def _edge_pass(h_hbm, econ_hbm, src_hbm, dst_hbm, out_hbm,
               buf, srci_v, dsti_v, agg_sp, sem, scsem):
    cid = lax.axis_index("c")
    sid = lax.axis_index("s")
    wid = sid * 2 + cid
    cb = wid * NCH_MIN + jnp.minimum(wid, NCH_REM)
    nch = NCH_MIN + jnp.where(wid < NCH_REM, 1, 0)

    zero = jnp.zeros((16,), jnp.float32)

    def zero_row(i, _):
        for k in range(8):
            buf[0][i, pl.ds(k * 16, 16)] = zero
        return 0

    lax.fori_loop(0, K, zero_row, 0)

    def _zero_slice(base, sizes):
        off = 0
        for sz in sizes:
            pltpu.sync_copy(buf[0].at[pl.ds(0, sz)],
                            agg_sp.at[pl.ds(base + off, sz)])
            off += sz

    @pl.when(sid < 15)
    def _():
        _zero_slice(sid * ROWS_MAIN, [128, 128, 128, 128, 120])

    @pl.when(sid == 15)
    def _():
        _zero_slice(15 * ROWS_MAIN, [128, 128, 128, 128, 8])

    plsc.subcore_barrier()

    def issue_loads(jj, s):
        pltpu.async_copy(econ_hbm.at[pl.ds((cb + jj) * K, K)], buf[s], sem[s])
        pltpu.async_copy(src_hbm.at[pl.ds((cb + jj) * K, K)], srci_v[s], sem[s])
        pltpu.async_copy(dst_hbm.at[pl.ds((cb + jj) * K, K)], dsti_v[s], sem[s])

    def wait_loads(s):
        pltpu.make_async_copy(econ_hbm.at[pl.ds(0, K)], buf[s], sem[s]).wait()
        pltpu.make_async_copy(src_hbm.at[pl.ds(0, K)], srci_v[s], sem[s]).wait()
        pltpu.make_async_copy(dst_hbm.at[pl.ds(0, K)], dsti_v[s], sem[s]).wait()

    def wait_gather(s):
        pltpu.make_async_copy(econ_hbm.at[pl.ds(0, K)], buf[s], sem[s]).wait()

    def wait_scatter(s):
        pltpu.make_async_copy(buf[s], agg_sp.at[pl.ds(0, K)], scsem[s]).wait()

    pltpu.sync_copy(econ_hbm.at[pl.ds(cb * K, K)], buf[0])
    pltpu.sync_copy(src_hbm.at[pl.ds(cb * K, K)], srci_v[0])
    pltpu.sync_copy(dst_hbm.at[pl.ds(cb * K, K)], dsti_v[0])
    pltpu.async_copy(h_hbm.at[srci_v[0]], buf[0], sem[0], add=True)
    issue_loads(1, 1)

    def group(g, _):
        for u in range(3):
            j = g * 3 + u
            s1 = (u + 1) % 3
            s2 = (u + 2) % 3

            @pl.when(j + 1 < nch)
            def _():
                wait_loads(s1)
                pltpu.async_copy(h_hbm.at[srci_v[s1]], buf[s1], sem[s1],
                                 add=True)

            @pl.when(j < nch)
            def _():
                wait_gather(u)

                def msg_row(i, _):
                    for r in range(4):
                        for k in range(8):
                            sl = pl.ds(k * 16, 16)
                            buf[u][i * 4 + r, sl] = jnp.maximum(
                                buf[u][i * 4 + r, sl], 0.0)
                    return 0

                lax.fori_loop(0, K // 4, msg_row, 0)
                pltpu.async_copy(buf[u], agg_sp.at[dsti_v[u]], scsem[u],
                                 add=True)

            @pl.when(jnp.logical_and(j >= 1, j + 2 < nch))
            def _():
                wait_scatter(s2)

            @pl.when(j + 2 < nch)
            def _():
                issue_loads(j + 2, s2)
        return 0

    lax.fori_loop(0, NGROUPS, group, 0)

    @pl.when(wid < NCH_REM)
    def _():
        for c in (NCH_MIN - 2, NCH_MIN - 1, NCH_MIN):
            wait_scatter(c % 3)

    @pl.when(wid >= NCH_REM)
    def _():
        for c in (NCH_MIN - 3, NCH_MIN - 2, NCH_MIN - 1):
            wait_scatter(c % 3)

    plsc.subcore_barrier()

    @pl.when(sid < 15)
    def _():
        row0 = sid * ROWS_MAIN
        pltpu.sync_copy(agg_sp.at[pl.ds(row0, ROWS_MAIN)],
                        out_hbm.at[pl.ds(cid * N + row0, ROWS_MAIN)])

    @pl.when(sid == 15)
    def _():
        row0 = 15 * ROWS_MAIN
        pltpu.sync_copy(agg_sp.at[pl.ds(row0, ROWS_TAIL)],
                        out_hbm.at[pl.ds(cid * N + row0, ROWS_TAIL)])


_RT_N = 1000
_GRID_N = N // _RT_N
_RT_E = 2000
_GRID_E = E // _RT_E


def _node_init_body(x_ref, ploop_ref, wn_ref, bn_ref, wl0_ref, h_ref):
    h = jnp.dot(x_ref[...], wn_ref[...], preferred_element_type=jnp.float32)
    h += jnp.dot(ploop_ref[...], wl0_ref[...], preferred_element_type=jnp.float32)
    h_ref[...] = h + bn_ref[...]


def _node_init(x, poly_loop, W_node, b_node, W_loop0):
    return pl.pallas_call(
        _node_init_body,
        grid=(_GRID_N,),
        in_specs=[
            pl.BlockSpec((_RT_N, D), lambda i: (i, 0)),
            pl.BlockSpec((_RT_N, EMB), lambda i: (i, 0)),
            pl.BlockSpec((D, D), lambda i: (0, 0)),
            pl.BlockSpec((1, D), lambda i: (0, 0)),
            pl.BlockSpec((EMB, D), lambda i: (0, 0)),
        ],
        out_specs=pl.BlockSpec((_RT_N, D), lambda i: (i, 0)),
        out_shape=jax.ShapeDtypeStruct((N, D), jnp.float32),
    )(x, poly_loop, W_node, b_node, W_loop0)


def _econ_body(ea_ref, pc_ref, we_ref, be_ref, wc_ref, e0_ref, e1_ref, e2_ref):
    pc = pc_ref[...]
    base = jnp.dot(ea_ref[...], we_ref[...], preferred_element_type=jnp.float32)
    base += be_ref[...]
    p0 = jnp.dot(pc, wc_ref[0], preferred_element_type=jnp.float32)
    p1 = jnp.dot(pc, wc_ref[1], preferred_element_type=jnp.float32)
    p2 = jnp.dot(pc, wc_ref[2], preferred_element_type=jnp.float32)
    m0 = (pc[:, 1] != 0).astype(jnp.float32)[:, None]
    m1 = (pc[:, 2] != 0).astype(jnp.float32)[:, None]
    e0 = base + m0 * p0
    e1 = e0 + m1 * p1
    e0_ref[...] = e0
    e1_ref[...] = e1
    e2_ref[...] = e1 + p2


def _econ(edge_attr, poly_conn, W_edge, b_edge, W_conn):
    out = jax.ShapeDtypeStruct((E, D), jnp.float32)
    return pl.pallas_call(
        _econ_body,
        grid=(_GRID_E,),
        in_specs=[
            pl.BlockSpec((_RT_E, DE), lambda i: (i, 0)),
            pl.BlockSpec((_RT_E, EMB), lambda i: (i, 0)),
            pl.BlockSpec((DE, D), lambda i: (0, 0)),
            pl.BlockSpec((1, D), lambda i: (0, 0)),
            pl.BlockSpec((3, EMB, D), lambda i: (0, 0, 0)),
        ],
        out_specs=[pl.BlockSpec((_RT_E, D), lambda i: (i, 0))] * 3,
        out_shape=[out, out, out],
    )(edge_attr, poly_conn, W_edge, b_edge, W_conn)


def _update_body(h_ref, agg_ref, w_ref, b_ref, ploop_ref, wl_ref, o_ref):
    agg = agg_ref[0] + agg_ref[1]
    u = jnp.dot(agg, w_ref[...], preferred_element_type=jnp.float32) + b_ref[...]
    h = h_ref[...] + jnp.maximum(u, 0.0)
    h += jnp.dot(ploop_ref[...], wl_ref[...], preferred_element_type=jnp.float32)
    o_ref[...] = h


def _update(h, agg2, W, b, poly_loop, wl):
    return pl.pallas_call(
        _update_body,
        grid=(_GRID_N,),
        in_specs=[
            pl.BlockSpec((_RT_N, D), lambda i: (i, 0)),
            pl.BlockSpec((2, _RT_N, D), lambda i: (0, i, 0)),
            pl.BlockSpec((D, D), lambda i: (0, 0)),
            pl.BlockSpec((1, D), lambda i: (0, 0)),
            pl.BlockSpec((_RT_N, EMB), lambda i: (i, 0)),
            pl.BlockSpec((EMB, D), lambda i: (0, 0)),
        ],
        out_specs=pl.BlockSpec((_RT_N, D), lambda i: (i, 0)),
        out_shape=jax.ShapeDtypeStruct((N, D), jnp.float32),
    )(h, agg2, W, b, poly_loop, wl)


def _final_body(h_ref, agg_ref, w_ref, b_ref, wo_ref, bo_ref, o_ref):
    agg = agg_ref[0] + agg_ref[1]
    u = jnp.dot(agg, w_ref[...], preferred_element_type=jnp.float32) + b_ref[...]
    h = h_ref[...] + jnp.maximum(u, 0.0)
    o_ref[...] = jnp.dot(h, wo_ref[...], preferred_element_type=jnp.float32) + bo_ref[...]


def _final(h, agg2, W, b, W_out, b_out):
    return pl.pallas_call(
        _final_body,
        grid=(_GRID_N,),
        in_specs=[
            pl.BlockSpec((_RT_N, D), lambda i: (i, 0)),
            pl.BlockSpec((2, _RT_N, D), lambda i: (0, i, 0)),
            pl.BlockSpec((D, D), lambda i: (0, 0)),
            pl.BlockSpec((1, D), lambda i: (0, 0)),
            pl.BlockSpec((D, DIM_OUT), lambda i: (0, 0)),
            pl.BlockSpec((1, DIM_OUT), lambda i: (0, 0)),
        ],
        out_specs=pl.BlockSpec((_RT_N, DIM_OUT), lambda i: (i, 0)),
        out_shape=jax.ShapeDtypeStruct((N, DIM_OUT), jnp.float32),
    )(h, agg2, W, b, W_out, b_out)


def kernel(x, edge_attr, edge_index, poly_loop, poly_index, poly_conn,
           W_node, b_node, W_edge, b_edge, W_loop, W_conn,
           W_msg, b_msg, W_out, b_out):
    del poly_index
    src = edge_index[0]
    dst = edge_index[1]

    h = _node_init(x, poly_loop, W_node, b_node.reshape(1, D), W_loop[0])
    e0, e1, e2 = _econ(edge_attr, poly_conn, W_edge, b_edge.reshape(1, D), W_conn)

    wl_zero = jnp.zeros((EMB, D), jnp.float32)
    wl_next = (None, W_loop[1], W_loop[2], None)
    econs = (e0, e0, e1, e2)
    for r in range(4):
        agg = _edge_pass(h, econs[r], src, dst).reshape(2, N, D)
        if r < 3:
            wl = wl_next[r] if wl_next[r] is not None else wl_zero
            h = _update(h, agg, W_msg[r], b_msg[r].reshape(1, D), poly_loop, wl)
        else:
            out = _final(h, agg, W_msg[r], b_msg[r].reshape(1, D),
                         W_out, b_out.reshape(1, DIM_OUT))
    return out


# speedup vs baseline: 1.0384x; 1.0010x over previous
import functools

import jax
import jax.numpy as jnp
from jax import lax
from jax.experimental import pallas as pl
from jax.experimental.pallas import tpu as pltpu
from jax.experimental.pallas import tpu_sc as plsc

N = 10000
E = 320000
D = 128
DE = 16
EMB = 9
DIM_OUT = 10

K = 128
NCHUNK = E // K
NW = 32
ROWS_MAIN = 632
ROWS_TAIL = N - 15 * ROWS_MAIN


_sc_mesh = plsc.VectorSubcoreMesh(core_axis_name="c", subcore_axis_name="s")


NCH_MIN = NCHUNK // NW
NCH_REM = NCHUNK - NCH_MIN * NW
NGROUPS = (NCH_MIN + 1 + 2) // 3 + 1


@functools.partial(
    pl.kernel,
    out_type=jax.ShapeDtypeStruct((2 * N, D), jnp.float32),
    mesh=_sc_mesh,
    scratch_types=[
        [pltpu.VMEM((K, D), jnp.float32) for _ in range(3)],
        [pltpu.VMEM((K,), jnp.int32) for _ in range(3)],
        [pltpu.VMEM((K,), jnp.int32) for _ in range(3)],
        pltpu.VMEM_SHARED((N, D), jnp.float32),
        [pltpu.SemaphoreType.DMA for _ in range(3)],
        [pltpu.SemaphoreType.DMA for _ in range(3)],
    ],
)
def _edge_pass(h_hbm, econ_hbm, ei_hbm, out_hbm,
               buf, srci_v, dsti_v, agg_sp, sem, scsem):
    cid = lax.axis_index("c")
    sid = lax.axis_index("s")
    wid = sid * 2 + cid
    cb = wid * NCH_MIN + jnp.minimum(wid, NCH_REM)
    nch = NCH_MIN + jnp.where(wid < NCH_REM, 1, 0)

    zero = jnp.zeros((16,), jnp.float32)

    def zero_row(i, _):
        for k in range(8):
            buf[0][i, pl.ds(k * 16, 16)] = zero
        return 0

    lax.fori_loop(0, K, zero_row, 0)

    def _zero_slice(base, sizes):
        off = 0
        for sz in sizes:
            pltpu.sync_copy(buf[0].at[pl.ds(0, sz)],
                            agg_sp.at[pl.ds(base + off, sz)])
            off += sz

    @pl.when(sid < 15)
    def _():
        _zero_slice(sid * ROWS_MAIN, [128, 128, 128, 128, 120])

    @pl.when(sid == 15)
    def _():
        _zero_slice(15 * ROWS_MAIN, [128, 128, 128, 128, 8])

    plsc.subcore_barrier()

    def issue_loads(jj, s):
        pltpu.async_copy(econ_hbm.at[pl.ds((cb + jj) * K, K)], buf[s], sem[s])
        pltpu.async_copy(ei_hbm.at[pl.ds((cb + jj) * K, K)], srci_v[s], sem[s])
        pltpu.async_copy(ei_hbm.at[pl.ds(E + (cb + jj) * K, K)], dsti_v[s],
                         sem[s])

    def wait_loads(s):
        pltpu.make_async_copy(econ_hbm.at[pl.ds(0, K)], buf[s], sem[s]).wait()
        pltpu.make_async_copy(ei_hbm.at[pl.ds(0, K)], srci_v[s], sem[s]).wait()
        pltpu.make_async_copy(ei_hbm.at[pl.ds(0, K)], dsti_v[s], sem[s]).wait()

    def wait_gather(s):
        pltpu.make_async_copy(econ_hbm.at[pl.ds(0, K)], buf[s], sem[s]).wait()

    def wait_scatter(s):
        pltpu.make_async_copy(buf[s], agg_sp.at[pl.ds(0, K)], scsem[s]).wait()

    pltpu.sync_copy(econ_hbm.at[pl.ds(cb * K, K)], buf[0])
    pltpu.sync_copy(ei_hbm.at[pl.ds(cb * K, K)], srci_v[0])
    pltpu.sync_copy(ei_hbm.at[pl.ds(E + cb * K, K)], dsti_v[0])
    pltpu.async_copy(h_hbm.at[srci_v[0]], buf[0], sem[0], add=True)
    issue_loads(1, 1)

    def group(g, _):
        for u in range(3):
            j = g * 3 + u
            s1 = (u + 1) % 3
            s2 = (u + 2) % 3

            @pl.when(j + 1 < nch)
            def _():
                wait_loads(s1)
                pltpu.async_copy(h_hbm.at[srci_v[s1]], buf[s1], sem[s1],
                                 add=True)

            @pl.when(j < nch)
            def _():
                wait_gather(u)

                def msg_row(i, _):
                    for r in range(4):
                        for k in range(8):
                            sl = pl.ds(k * 16, 16)
                            buf[u][i * 4 + r, sl] = jnp.maximum(
                                buf[u][i * 4 + r, sl], 0.0)
                    return 0

                lax.fori_loop(0, K // 4, msg_row, 0)
                pltpu.async_copy(buf[u], agg_sp.at[dsti_v[u]], scsem[u],
                                 add=True)

            @pl.when(jnp.logical_and(j >= 1, j + 2 < nch))
            def _():
                wait_scatter(s2)

            @pl.when(j + 2 < nch)
            def _():
                issue_loads(j + 2, s2)
        return 0

    lax.fori_loop(0, NGROUPS, group, 0)

    @pl.when(wid < NCH_REM)
    def _():
        for c in (NCH_MIN - 2, NCH_MIN - 1, NCH_MIN):
            wait_scatter(c % 3)

    @pl.when(wid >= NCH_REM)
    def _():
        for c in (NCH_MIN - 3, NCH_MIN - 2, NCH_MIN - 1):
            wait_scatter(c % 3)

    plsc.subcore_barrier()

    @pl.when(sid < 15)
    def _():
        row0 = sid * ROWS_MAIN
        pltpu.sync_copy(agg_sp.at[pl.ds(row0, ROWS_MAIN)],
                        out_hbm.at[pl.ds(cid * N + row0, ROWS_MAIN)])

    @pl.when(sid == 15)
    def _():
        row0 = 15 * ROWS_MAIN
        pltpu.sync_copy(agg_sp.at[pl.ds(row0, ROWS_TAIL)],
                        out_hbm.at[pl.ds(cid * N + row0, ROWS_TAIL)])


_RT_N = 1000
_GRID_N = N // _RT_N
_RT_E = 2000
_GRID_E = E // _RT_E


def _node_init_body(x_ref, ploop_ref, wn_ref, bn_ref, wl0_ref, h_ref):
    h = jnp.dot(x_ref[...], wn_ref[...], preferred_element_type=jnp.float32)
    h += jnp.dot(ploop_ref[...], wl0_ref[...], preferred_element_type=jnp.float32)
    h_ref[...] = h + bn_ref[...]


def _node_init(x, poly_loop, W_node, b_node, W_loop0):
    return pl.pallas_call(
        _node_init_body,
        grid=(_GRID_N,),
        in_specs=[
            pl.BlockSpec((_RT_N, D), lambda i: (i, 0)),
            pl.BlockSpec((_RT_N, EMB), lambda i: (i, 0)),
            pl.BlockSpec((D, D), lambda i: (0, 0)),
            pl.BlockSpec((1, D), lambda i: (0, 0)),
            pl.BlockSpec((EMB, D), lambda i: (0, 0)),
        ],
        out_specs=pl.BlockSpec((_RT_N, D), lambda i: (i, 0)),
        out_shape=jax.ShapeDtypeStruct((N, D), jnp.float32),
    )(x, poly_loop, W_node, b_node, W_loop0)


def _econ_body(ea_ref, pc_ref, we_ref, be_ref, wc_ref, e0_ref, e1_ref, e2_ref):
    pc = pc_ref[...]
    base = jnp.dot(ea_ref[...], we_ref[...], preferred_element_type=jnp.float32)
    base += be_ref[...]
    p0 = jnp.dot(pc, wc_ref[0], preferred_element_type=jnp.float32)
    p1 = jnp.dot(pc, wc_ref[1], preferred_element_type=jnp.float32)
    p2 = jnp.dot(pc, wc_ref[2], preferred_element_type=jnp.float32)
    m0 = (pc[:, 1] != 0).astype(jnp.float32)[:, None]
    m1 = (pc[:, 2] != 0).astype(jnp.float32)[:, None]
    e0 = base + m0 * p0
    e1 = e0 + m1 * p1
    e0_ref[...] = e0
    e1_ref[...] = e1
    e2_ref[...] = e1 + p2


def _econ(edge_attr, poly_conn, W_edge, b_edge, W_conn):
    out = jax.ShapeDtypeStruct((E, D), jnp.float32)
    return pl.pallas_call(
        _econ_body,
        grid=(_GRID_E,),
        in_specs=[
            pl.BlockSpec((_RT_E, DE), lambda i: (i, 0)),
            pl.BlockSpec((_RT_E, EMB), lambda i: (i, 0)),
            pl.BlockSpec((DE, D), lambda i: (0, 0)),
            pl.BlockSpec((1, D), lambda i: (0, 0)),
            pl.BlockSpec((3, EMB, D), lambda i: (0, 0, 0)),
        ],
        out_specs=[pl.BlockSpec((_RT_E, D), lambda i: (i, 0))] * 3,
        out_shape=[out, out, out],
    )(edge_attr, poly_conn, W_edge, b_edge, W_conn)


def _update_body(h_ref, agg_ref, w_ref, b_ref, ploop_ref, wl_ref, o_ref):
    agg = agg_ref[0] + agg_ref[1]
    u = jnp.dot(agg, w_ref[...], preferred_element_type=jnp.float32) + b_ref[...]
    h = h_ref[...] + jnp.maximum(u, 0.0)
    h += jnp.dot(ploop_ref[...], wl_ref[...], preferred_element_type=jnp.float32)
    o_ref[...] = h


def _update(h, agg2, W, b, poly_loop, wl):
    return pl.pallas_call(
        _update_body,
        grid=(_GRID_N,),
        in_specs=[
            pl.BlockSpec((_RT_N, D), lambda i: (i, 0)),
            pl.BlockSpec((2, _RT_N, D), lambda i: (0, i, 0)),
            pl.BlockSpec((D, D), lambda i: (0, 0)),
            pl.BlockSpec((1, D), lambda i: (0, 0)),
            pl.BlockSpec((_RT_N, EMB), lambda i: (i, 0)),
            pl.BlockSpec((EMB, D), lambda i: (0, 0)),
        ],
        out_specs=pl.BlockSpec((_RT_N, D), lambda i: (i, 0)),
        out_shape=jax.ShapeDtypeStruct((N, D), jnp.float32),
    )(h, agg2, W, b, poly_loop, wl)


def _final_body(h_ref, agg_ref, w_ref, b_ref, wo_ref, bo_ref, o_ref):
    agg = agg_ref[0] + agg_ref[1]
    u = jnp.dot(agg, w_ref[...], preferred_element_type=jnp.float32) + b_ref[...]
    h = h_ref[...] + jnp.maximum(u, 0.0)
    o_ref[...] = jnp.dot(h, wo_ref[...], preferred_element_type=jnp.float32) + bo_ref[...]


def _final(h, agg2, W, b, W_out, b_out):
    return pl.pallas_call(
        _final_body,
        grid=(_GRID_N,),
        in_specs=[
            pl.BlockSpec((_RT_N, D), lambda i: (i, 0)),
            pl.BlockSpec((2, _RT_N, D), lambda i: (0, i, 0)),
            pl.BlockSpec((D, D), lambda i: (0, 0)),
            pl.BlockSpec((1, D), lambda i: (0, 0)),
            pl.BlockSpec((D, DIM_OUT), lambda i: (0, 0)),
            pl.BlockSpec((1, DIM_OUT), lambda i: (0, 0)),
        ],
        out_specs=pl.BlockSpec((_RT_N, DIM_OUT), lambda i: (i, 0)),
        out_shape=jax.ShapeDtypeStruct((N, DIM_OUT), jnp.float32),
    )(h, agg2, W, b, W_out, b_out)


def kernel(x, edge_attr, edge_index, poly_loop, poly_index, poly_conn,
           W_node, b_node, W_edge, b_edge, W_loop, W_conn,
           W_msg, b_msg, W_out, b_out):
    del poly_index
    ei = edge_index.reshape(2 * E)

    h = _node_init(x, poly_loop, W_node, b_node.reshape(1, D), W_loop[0])
    e0, e1, e2 = _econ(edge_attr, poly_conn, W_edge, b_edge.reshape(1, D), W_conn)

    wl_zero = jnp.zeros((EMB, D), jnp.float32)
    wl_next = (None, W_loop[1], W_loop[2], None)
    econs = (e0, e0, e1, e2)
    for r in range(4):
        agg = _edge_pass(h, econs[r], ei).reshape(2, N, D)
        if r < 3:
            wl = wl_next[r] if wl_next[r] is not None else wl_zero
            h = _update(h, agg, W_msg[r], b_msg[r].reshape(1, D), poly_loop, wl)
        else:
            out = _final(h, agg, W_msg[r], b_msg[r].reshape(1, D),
                         W_out, b_out.reshape(1, DIM_OUT))
    return out
